# baseline (speedup 1.0000x reference)
import jax
import jax.numpy as jnp
from jax import lax
from jax.experimental import pallas as pl
from jax.experimental.pallas import tpu as pltpu
from jax.experimental.pallas import tpu_sc as plsc

DIM = 32
BATCH = 16384
VOCAB = 1000000
LANE = 128
PACK = 4
UBLK = PACK * LANE
N_BLOCKS = (VOCAB + UBLK - 1) // UBLK
PACKED_ROWS = N_BLOCKS * LANE

NUM_WORKERS = 32
B_PER_W = BATCH // NUM_WORKERS
CHUNK = 256
N_CHUNKS = B_PER_W // CHUNK
IDX_BLK = 128
BLK_PER_CHUNK = CHUNK // IDX_BLK
LANES = 16
GROUPS = CHUNK // LANES


SUB = 128


def _placement(q):
    e = jnp.eye(DIM, dtype=jnp.float32)
    return jnp.pad(e, ((0, 0), (DIM * q, LANE - DIM * (q + 1))))


def _pack_body(x_ref, o_ref):
    es = [_placement(q) for q in range(PACK)]
    for b in range(SUB):
        y = jnp.zeros((LANE, LANE), jnp.float32)
        for q in range(PACK):
            xs = x_ref[:, pl.ds(UBLK * b + LANE * q, LANE)]
            y = y + lax.dot_general(
                xs, es[q], (((0,), (0,)), ((), ())),
                preferred_element_type=jnp.float32)
        o_ref[LANE * b:LANE * (b + 1), :] = y


def _tc_pack(table):
    tt = table.T
    return pl.pallas_call(
        _pack_body,
        grid=(pl.cdiv(N_BLOCKS, SUB),),
        in_specs=[pl.BlockSpec((DIM, UBLK * SUB), lambda i: (0, i))],
        out_specs=pl.BlockSpec((LANE * SUB, LANE), lambda i: (i, 0)),
        out_shape=jax.ShapeDtypeStruct((pl.cdiv(N_BLOCKS, SUB) * SUB * LANE,
                                        LANE), jnp.float32),
    )(tt)


def _dot_body(user_hbm, item_hbm, utab_hbm, itab_hbm, out_hbm,
              uidx_v, iidx_v, upid_v, ipid_v, urows_v, irows_v, out_v, sem):
    wid = lax.axis_index("s") * 2 + lax.axis_index("c")
    base = wid * B_PER_W

    iota = lax.iota(jnp.int32, LANES)

    def packed_row(u):
        return lax.shift_left(lax.shift_right_logical(u, 9), 7) + \
            jnp.bitwise_and(u, 127)

    def col_base(u):
        return lax.shift_left(
            jnp.bitwise_and(lax.shift_right_logical(u, 7), 3), 5)

    for ch in range(N_CHUNKS):
        off = base + ch * CHUNK
        for j in range(BLK_PER_CHUNK):
            pltpu.sync_copy(user_hbm.at[pl.ds(off + j * IDX_BLK, IDX_BLK)],
                            uidx_v.at[j])
            pltpu.sync_copy(item_hbm.at[pl.ds(off + j * IDX_BLK, IDX_BLK)],
                            iidx_v.at[j])

        def pid_body(g, carry):
            j = g // (IDX_BLK // LANES)
            lane0 = (g % (IDX_BLK // LANES)) * LANES
            sl = pl.ds(lane0, LANES)
            upid_v[j, sl] = packed_row(uidx_v[j, sl])
            ipid_v[j, sl] = packed_row(iidx_v[j, sl])
            return carry

        lax.fori_loop(0, CHUNK // LANES, pid_body, 0)

        copies = []
        for j in range(BLK_PER_CHUNK):
            dst = pl.ds(j * IDX_BLK, IDX_BLK)
            copies.append(pltpu.async_copy(utab_hbm.at[upid_v.at[j]],
                                           urows_v.at[dst], sem))
            copies.append(pltpu.async_copy(itab_hbm.at[ipid_v.at[j]],
                                           irows_v.at[dst], sem))
        for c in copies:
            c.wait()

        def group_body(g, carry):
            rows = g * LANES + iota
            j = g // (IDX_BLK // LANES)
            lane0 = (g % (IDX_BLK // LANES)) * LANES
            sl = pl.ds(lane0, LANES)
            ucol0 = col_base(uidx_v[j, sl])
            icol0 = col_base(iidx_v[j, sl])
            acc = jnp.zeros((LANES,), jnp.float32)
            for d in range(DIM):
                uc = plsc.load_gather(urows_v, [rows, ucol0 + d])
                ic = plsc.load_gather(irows_v, [rows, icol0 + d])
                acc = acc + uc * ic
            out_v[pl.ds(ch * CHUNK + g * LANES, LANES)] = acc
            return carry

        lax.fori_loop(0, GROUPS, group_body, 0)

    pltpu.sync_copy(out_v, out_hbm.at[pl.ds(base, B_PER_W)])


def _sc_dot(user, item, utab, itab):
    mesh = plsc.VectorSubcoreMesh(core_axis_name="c", subcore_axis_name="s")
    return pl.kernel(
        _dot_body,
        out_type=jax.ShapeDtypeStruct((BATCH,), jnp.float32),
        mesh=mesh,
        compiler_params=pltpu.CompilerParams(
            needs_layout_passes=False,
            use_tc_tiling_on_sc=True,
        ),
        scratch_types=[
            pltpu.VMEM((BLK_PER_CHUNK, IDX_BLK), jnp.int32),
            pltpu.VMEM((BLK_PER_CHUNK, IDX_BLK), jnp.int32),
            pltpu.VMEM((BLK_PER_CHUNK, IDX_BLK), jnp.int32),
            pltpu.VMEM((BLK_PER_CHUNK, IDX_BLK), jnp.int32),
            pltpu.VMEM((CHUNK, LANE), jnp.float32),
            pltpu.VMEM((CHUNK, LANE), jnp.float32),
            pltpu.VMEM((B_PER_W,), jnp.float32),
            pltpu.SemaphoreType.DMA,
        ],
    )(user, item, utab, itab)


@jax.jit
def _mf(user, item, user_table, item_table):
    utab = _tc_pack(user_table)
    itab = _tc_pack(item_table)
    return _sc_dot(user, item, utab, itab)


def kernel(user, item, user_table, item_table):
    return _mf(user, item, user_table, item_table)


# speedup vs baseline: 1.0187x; 1.0187x over previous
import jax
import jax.numpy as jnp
from jax import lax
from jax.experimental import pallas as pl
from jax.experimental.pallas import tpu as pltpu
from jax.experimental.pallas import tpu_sc as plsc

DIM = 32
BATCH = 16384
VOCAB = 1000000
LANE = 128
PACK = 4
UBLK = PACK * LANE
N_BLOCKS = (VOCAB + UBLK - 1) // UBLK
PACKED_ROWS = N_BLOCKS * LANE

NUM_WORKERS = 32
B_PER_W = BATCH // NUM_WORKERS
CHUNK = 256
N_CHUNKS = B_PER_W // CHUNK
IDX_BLK = 128
BLK_PER_CHUNK = CHUNK // IDX_BLK
LANES = 16
GROUPS = CHUNK // LANES


SUB = 64


def _placement(q):
    e = jnp.eye(DIM, dtype=jnp.float32)
    return jnp.pad(e, ((0, 0), (DIM * q, LANE - DIM * (q + 1))))


def _pack_body(x_ref, o_ref):
    es = [_placement(q) for q in range(PACK)]
    for b in range(SUB):
        y = jnp.zeros((LANE, LANE), jnp.float32)
        for q in range(PACK):
            xs = x_ref[:, pl.ds(UBLK * b + LANE * q, LANE)]
            y = y + lax.dot_general(
                xs, es[q], (((0,), (0,)), ((), ())),
                preferred_element_type=jnp.float32)
        o_ref[LANE * b:LANE * (b + 1), :] = y


def _tc_pack(table):
    tt = table.T
    return pl.pallas_call(
        _pack_body,
        grid=(pl.cdiv(N_BLOCKS, SUB),),
        in_specs=[pl.BlockSpec((DIM, UBLK * SUB), lambda i: (0, i))],
        out_specs=pl.BlockSpec((LANE * SUB, LANE), lambda i: (i, 0)),
        out_shape=jax.ShapeDtypeStruct((pl.cdiv(N_BLOCKS, SUB) * SUB * LANE,
                                        LANE), jnp.float32),
    )(tt)


def _dot_body(user_hbm, item_hbm, utab_hbm, itab_hbm, out_hbm,
              uidx_v, iidx_v, upid_v, ipid_v, urows_v, irows_v, out_v, sem):
    wid = lax.axis_index("s") * 2 + lax.axis_index("c")
    base = wid * B_PER_W

    iota = lax.iota(jnp.int32, LANES)

    def packed_row(u):
        return lax.shift_left(lax.shift_right_logical(u, 9), 7) + \
            jnp.bitwise_and(u, 127)

    def col_base(u):
        return lax.shift_left(
            jnp.bitwise_and(lax.shift_right_logical(u, 7), 3), 5)

    for ch in range(N_CHUNKS):
        off = base + ch * CHUNK
        for j in range(BLK_PER_CHUNK):
            pltpu.sync_copy(user_hbm.at[pl.ds(off + j * IDX_BLK, IDX_BLK)],
                            uidx_v.at[j])
            pltpu.sync_copy(item_hbm.at[pl.ds(off + j * IDX_BLK, IDX_BLK)],
                            iidx_v.at[j])

        def pid_body(g, carry):
            j = g // (IDX_BLK // LANES)
            lane0 = (g % (IDX_BLK // LANES)) * LANES
            sl = pl.ds(lane0, LANES)
            upid_v[j, sl] = packed_row(uidx_v[j, sl])
            ipid_v[j, sl] = packed_row(iidx_v[j, sl])
            return carry

        lax.fori_loop(0, CHUNK // LANES, pid_body, 0)

        copies = []
        for j in range(BLK_PER_CHUNK):
            dst = pl.ds(j * IDX_BLK, IDX_BLK)
            copies.append(pltpu.async_copy(utab_hbm.at[upid_v.at[j]],
                                           urows_v.at[dst], sem))
            copies.append(pltpu.async_copy(itab_hbm.at[ipid_v.at[j]],
                                           irows_v.at[dst], sem))
        for c in copies:
            c.wait()

        def group_body(g, carry):
            rows = g * LANES + iota
            j = g // (IDX_BLK // LANES)
            lane0 = (g % (IDX_BLK // LANES)) * LANES
            sl = pl.ds(lane0, LANES)
            ucol0 = col_base(uidx_v[j, sl])
            icol0 = col_base(iidx_v[j, sl])
            acc = jnp.zeros((LANES,), jnp.float32)
            for d in range(DIM):
                uc = plsc.load_gather(urows_v, [rows, ucol0 + d])
                ic = plsc.load_gather(irows_v, [rows, icol0 + d])
                acc = acc + uc * ic
            out_v[pl.ds(ch * CHUNK + g * LANES, LANES)] = acc
            return carry

        lax.fori_loop(0, GROUPS, group_body, 0)

    pltpu.sync_copy(out_v, out_hbm.at[pl.ds(base, B_PER_W)])


def _sc_dot(user, item, utab, itab):
    mesh = plsc.VectorSubcoreMesh(core_axis_name="c", subcore_axis_name="s")
    return pl.kernel(
        _dot_body,
        out_type=jax.ShapeDtypeStruct((BATCH,), jnp.float32),
        mesh=mesh,
        compiler_params=pltpu.CompilerParams(
            needs_layout_passes=False,
            use_tc_tiling_on_sc=True,
        ),
        scratch_types=[
            pltpu.VMEM((BLK_PER_CHUNK, IDX_BLK), jnp.int32),
            pltpu.VMEM((BLK_PER_CHUNK, IDX_BLK), jnp.int32),
            pltpu.VMEM((BLK_PER_CHUNK, IDX_BLK), jnp.int32),
            pltpu.VMEM((BLK_PER_CHUNK, IDX_BLK), jnp.int32),
            pltpu.VMEM((CHUNK, LANE), jnp.float32),
            pltpu.VMEM((CHUNK, LANE), jnp.float32),
            pltpu.VMEM((B_PER_W,), jnp.float32),
            pltpu.SemaphoreType.DMA,
        ],
    )(user, item, utab, itab)


@jax.jit
def _mf(user, item, user_table, item_table):
    utab = _tc_pack(user_table)
    itab = _tc_pack(item_table)
    return _sc_dot(user, item, utab, itab)


def kernel(user, item, user_table, item_table):
    return _mf(user, item, user_table, item_table)
